# Initial kernel scaffold; baseline (speedup 1.0000x reference)
import jax
import jax.numpy as jnp
from jax.experimental import pallas as pl


def kernel(x, edge_index, W0, b0, W1, b1, W2, b2, Wl, bl):
    raise NotImplementedError("write your pallas kernel here")


# speedup vs baseline: 7.7455x; 7.7455x over previous
import functools

import jax
import jax.numpy as jnp
from jax import lax
from jax.experimental import pallas as pl
from jax.experimental.pallas import tpu as pltpu
from jax.experimental.pallas import tpu_sc as plsc

NN = 10000
EE = 320000
F_IN = 128
HID = 256
HALF = 128

NC = 2
NS = 16
ROWS_PER_TILE = NN // NS


def _zero_fill(zbuf, nrows, ncols):
    per_row = ncols // 16

    def body(i, _):
        r = i // per_row
        j = (i % per_row) * 16
        zbuf[r, pl.ds(j, 16)] = jnp.zeros((16,), jnp.float32)
        return 0

    lax.fori_loop(0, nrows * per_row, body, 0)


_DEG_BK = 80
_DEG_EPC = EE // NC
_DEG_EPT = _DEG_EPC // NS
_DEG_NB = _DEG_EPT // _DEG_BK


@functools.cache
def _get_deg_kernel():
    mesh = plsc.VectorSubcoreMesh(core_axis_name="c", subcore_axis_name="s",
                                  num_cores=NC, num_subcores=NS)
    return pl.kernel(
        _deg_body,
        out_type=jax.ShapeDtypeStruct((NC, NS, ROWS_PER_TILE, 16), jnp.float32),
        mesh=mesh,
        scratch_types=[
            pltpu.VMEM((_DEG_BK,), jnp.int32),
            pltpu.VMEM((_DEG_BK, 16), jnp.float32),
            pltpu.VMEM((125, 16), jnp.float32),
            pltpu.VMEM_SHARED((NN, 16), jnp.float32),
        ],
    )


def _deg_body(dst_hbm, out_hbm, idx_v, ones_v, zbuf, acc):
    c = lax.axis_index("c")
    s = lax.axis_index("s")

    def fill_ones(i, _):
        ones_v[i, :] = jnp.ones((16,), jnp.float32)
        return 0

    lax.fori_loop(0, _DEG_BK, fill_ones, 0)
    _zero_fill(zbuf, 125, 16)

    def zcopy(i, _):
        pltpu.sync_copy(zbuf, acc.at[pl.ds(s * ROWS_PER_TILE + i * 125, 125)])
        return 0

    lax.fori_loop(0, 5, zcopy, 0)
    plsc.subcore_barrier()

    base = c * _DEG_EPC + s * _DEG_EPT

    def batch(i, _):
        pltpu.sync_copy(dst_hbm.at[pl.ds(base + i * _DEG_BK, _DEG_BK)], idx_v)
        pltpu.sync_copy(ones_v, acc.at[idx_v], add=True)
        return 0

    lax.fori_loop(0, _DEG_NB, batch, 0)
    plsc.subcore_barrier()

    pltpu.sync_copy(acc.at[pl.ds(s * ROWS_PER_TILE, ROWS_PER_TILE)],
                    out_hbm.at[c].at[s])


_AGG_BK = 80
_AGG_EPT = EE // NS
_AGG_NB = _AGG_EPT // _AGG_BK


@functools.cache
def _get_agg_kernel():
    mesh = plsc.VectorSubcoreMesh(core_axis_name="c", subcore_axis_name="s",
                                  num_cores=NC, num_subcores=NS)
    return pl.kernel(
        _agg_body,
        out_type=jax.ShapeDtypeStruct((NC, NS, ROWS_PER_TILE, HALF), jnp.float32),
        mesh=mesh,
        scratch_types=[
            pltpu.VMEM((_AGG_BK,), jnp.int32),
            pltpu.VMEM((_AGG_BK,), jnp.int32),
            pltpu.VMEM((_AGG_BK, HALF), jnp.float32),
            pltpu.VMEM((25, HALF), jnp.float32),
            pltpu.VMEM_SHARED((NN, HALF), jnp.float32),
            pltpu.SemaphoreType.DMA,
        ],
    )


def _agg_body(hs_lo_hbm, hs_hi_hbm, src_hbm, dst_hbm, out_hbm,
              idx_s, idx_d, rows, zbuf, acc, sem):
    c = lax.axis_index("c")
    s = lax.axis_index("s")

    _zero_fill(zbuf, 25, HALF)

    def zcopy(i, _):
        pltpu.sync_copy(zbuf, acc.at[pl.ds(s * ROWS_PER_TILE + i * 25, 25)])
        return 0

    lax.fori_loop(0, 25, zcopy, 0)
    plsc.subcore_barrier()

    base = s * _AGG_EPT

    def run_edges(hs_ref):
        def batch(i, _):
            off = base + i * _AGG_BK
            pltpu.sync_copy(src_hbm.at[pl.ds(off, _AGG_BK)], idx_s)
            pltpu.sync_copy(dst_hbm.at[pl.ds(off, _AGG_BK)], idx_d)
            pltpu.async_copy(hs_ref.at[idx_s], rows, sem).wait()
            pltpu.sync_copy(rows, acc.at[idx_d], add=True)
            return 0

        lax.fori_loop(0, _AGG_NB, batch, 0)

    pl.when(c == 0)(lambda: run_edges(hs_lo_hbm))
    pl.when(c == 1)(lambda: run_edges(hs_hi_hbm))
    plsc.subcore_barrier()

    pltpu.sync_copy(acc.at[pl.ds(s * ROWS_PER_TILE, ROWS_PER_TILE)],
                    out_hbm.at[c].at[s])


_BN = 2000


def _a0_body(deg_ref, x_ref, w_ref, lo_ref, hi_ref, dinv_ref):
    d = deg_ref[0, :, 0] + deg_ref[1, :, 0] + 1.0
    dv = lax.rsqrt(d)
    hw = jnp.dot(x_ref[...], w_ref[...], preferred_element_type=jnp.float32)
    hs = hw * dv[:, None]
    lo_ref[...] = hs[:, :HALF]
    hi_ref[...] = hs[:, HALF:]
    dinv_ref[...] = dv[:, None]


def _layer0(degs, x, w0):
    return pl.pallas_call(
        _a0_body,
        grid=(NN // _BN,),
        in_specs=[
            pl.BlockSpec((NC, _BN, 16), lambda i: (0, i, 0)),
            pl.BlockSpec((_BN, F_IN), lambda i: (i, 0)),
            pl.BlockSpec((F_IN, HID), lambda i: (0, 0)),
        ],
        out_specs=[
            pl.BlockSpec((_BN, HALF), lambda i: (i, 0)),
            pl.BlockSpec((_BN, HALF), lambda i: (i, 0)),
            pl.BlockSpec((_BN, 1), lambda i: (i, 0)),
        ],
        out_shape=[
            jax.ShapeDtypeStruct((NN, HALF), jnp.float32),
            jax.ShapeDtypeStruct((NN, HALF), jnp.float32),
            jax.ShapeDtypeStruct((NN, 1), jnp.float32),
        ],
    )(degs, x, w0)


def _amid_body(agg_ref, lo_ref, hi_ref, dinv_ref, b_ref, w_ref,
               olo_ref, ohi_ref):
    dv = dinv_ref[...]
    lo = jax.nn.relu((agg_ref[0] + lo_ref[...]) * dv + b_ref[0, :HALF][None, :])
    hi = jax.nn.relu((agg_ref[1] + hi_ref[...]) * dv + b_ref[0, HALF:][None, :])
    h = jnp.concatenate([lo, hi], axis=1)
    hs = jnp.dot(h, w_ref[...], preferred_element_type=jnp.float32) * dv
    olo_ref[...] = hs[:, :HALF]
    ohi_ref[...] = hs[:, HALF:]


def _layer_mid(agg, hs_lo, hs_hi, dinv, b_prev, w):
    return pl.pallas_call(
        _amid_body,
        grid=(NN // _BN,),
        in_specs=[
            pl.BlockSpec((NC, _BN, HALF), lambda i: (0, i, 0)),
            pl.BlockSpec((_BN, HALF), lambda i: (i, 0)),
            pl.BlockSpec((_BN, HALF), lambda i: (i, 0)),
            pl.BlockSpec((_BN, 1), lambda i: (i, 0)),
            pl.BlockSpec((1, HID), lambda i: (0, 0)),
            pl.BlockSpec((HID, HID), lambda i: (0, 0)),
        ],
        out_specs=[
            pl.BlockSpec((_BN, HALF), lambda i: (i, 0)),
            pl.BlockSpec((_BN, HALF), lambda i: (i, 0)),
        ],
        out_shape=[
            jax.ShapeDtypeStruct((NN, HALF), jnp.float32),
            jax.ShapeDtypeStruct((NN, HALF), jnp.float32),
        ],
    )(agg, hs_lo, hs_hi, dinv, b_prev, w)


def _a3_body(agg_ref, lo_ref, hi_ref, dinv_ref, b_ref, wl_ref, bl_ref,
             out_ref):
    dv = dinv_ref[...]
    lo = jax.nn.relu((agg_ref[0] + lo_ref[...]) * dv + b_ref[0, :HALF][None, :])
    hi = jax.nn.relu((agg_ref[1] + hi_ref[...]) * dv + b_ref[0, HALF:][None, :])
    h = jnp.concatenate([lo, hi], axis=1)
    out_ref[...] = (
        jnp.dot(h, wl_ref[...], preferred_element_type=jnp.float32)
        + bl_ref[0, 0]
    )


def _layer_final(agg, hs_lo, hs_hi, dinv, b2, wl, bl):
    return pl.pallas_call(
        _a3_body,
        grid=(NN // _BN,),
        in_specs=[
            pl.BlockSpec((NC, _BN, HALF), lambda i: (0, i, 0)),
            pl.BlockSpec((_BN, HALF), lambda i: (i, 0)),
            pl.BlockSpec((_BN, HALF), lambda i: (i, 0)),
            pl.BlockSpec((_BN, 1), lambda i: (i, 0)),
            pl.BlockSpec((1, HID), lambda i: (0, 0)),
            pl.BlockSpec((HID, 1), lambda i: (0, 0)),
            pl.BlockSpec((1, 1), lambda i: (0, 0)),
        ],
        out_specs=pl.BlockSpec((_BN, 1), lambda i: (i, 0)),
        out_shape=jax.ShapeDtypeStruct((NN, 1), jnp.float32),
    )(agg, hs_lo, hs_hi, dinv, b2, wl, bl)


def kernel(x, edge_index, W0, b0, W1, b1, W2, b2, Wl, bl):
    src = edge_index[0].astype(jnp.int32)
    dst = edge_index[1].astype(jnp.int32)

    degs = _get_deg_kernel()(dst).reshape(NC, NN, 16)
    hs_lo, hs_hi, dinv = _layer0(degs, x, W0)

    agg = _get_agg_kernel()(hs_lo, hs_hi, src, dst).reshape(NC, NN, HALF)
    hs_lo, hs_hi = _layer_mid(agg, hs_lo, hs_hi, dinv, b0.reshape(1, -1), W1)

    agg = _get_agg_kernel()(hs_lo, hs_hi, src, dst).reshape(NC, NN, HALF)
    hs_lo, hs_hi = _layer_mid(agg, hs_lo, hs_hi, dinv, b1.reshape(1, -1), W2)

    agg = _get_agg_kernel()(hs_lo, hs_hi, src, dst).reshape(NC, NN, HALF)
    out = _layer_final(agg, hs_lo, hs_hi, dinv, b2.reshape(1, -1), Wl,
                       bl.reshape(1, 1))
    return out.reshape(-1)


# speedup vs baseline: 17.8202x; 2.3007x over previous
import functools

import jax
import jax.numpy as jnp
from jax import lax
from jax.experimental import pallas as pl
from jax.experimental.pallas import tpu as pltpu
from jax.experimental.pallas import tpu_sc as plsc

NN = 10000
EE = 320000
F_IN = 128
HID = 256
HALF = 128

NC = 2
NS = 16
ROWS_PER_TILE = NN // NS


def _zero_fill(zbuf, nrows, ncols):
    per_row = ncols // 16

    def body(i, _):
        r = i // per_row
        j = (i % per_row) * 16
        zbuf[r, pl.ds(j, 16)] = jnp.zeros((16,), jnp.float32)
        return 0

    lax.fori_loop(0, nrows * per_row, body, 0)


_BK = 125
_NBLK = EE // _BK
_DEG_BPT = _NBLK // (NC * NS)


@functools.cache
def _get_deg_kernel():
    mesh = plsc.VectorSubcoreMesh(core_axis_name="c", subcore_axis_name="s",
                                  num_cores=NC, num_subcores=NS)
    return pl.kernel(
        _deg_body,
        out_type=jax.ShapeDtypeStruct((NC, NS, ROWS_PER_TILE, 16), jnp.float32),
        mesh=mesh,
        scratch_types=[
            pltpu.VMEM((2, _BK), jnp.int32),
            pltpu.VMEM((_BK, 16), jnp.float32),
            pltpu.VMEM((125, 16), jnp.float32),
            pltpu.VMEM_SHARED((NN, 16), jnp.float32),
        ],
    )


def _deg_body(eb_hbm, out_hbm, idx_v, ones_v, zbuf, acc):
    c = lax.axis_index("c")
    s = lax.axis_index("s")

    def fill_ones(i, _):
        ones_v[i, :] = jnp.ones((16,), jnp.float32)
        return 0

    lax.fori_loop(0, _BK, fill_ones, 0)
    _zero_fill(zbuf, 125, 16)

    def zcopy(i, _):
        pltpu.sync_copy(zbuf, acc.at[pl.ds(s * ROWS_PER_TILE + i * 125, 125)])
        return 0

    lax.fori_loop(0, 5, zcopy, 0)
    plsc.subcore_barrier()

    base = (c * NS + s) * _DEG_BPT

    def batch(i, _):
        pltpu.sync_copy(eb_hbm.at[base + i], idx_v)
        pltpu.sync_copy(ones_v, acc.at[idx_v.at[1]], add=True)
        return 0

    lax.fori_loop(0, _DEG_BPT, batch, 0)
    plsc.subcore_barrier()

    pltpu.sync_copy(acc.at[pl.ds(s * ROWS_PER_TILE, ROWS_PER_TILE)],
                    out_hbm.at[c].at[s])


_AGG_BPT = _NBLK // NS


@functools.cache
def _get_agg_kernel():
    mesh = plsc.VectorSubcoreMesh(core_axis_name="c", subcore_axis_name="s",
                                  num_cores=NC, num_subcores=NS)
    return pl.kernel(
        _agg_body,
        out_type=jax.ShapeDtypeStruct((NC, NS, ROWS_PER_TILE, HALF), jnp.float32),
        mesh=mesh,
        scratch_types=[
            pltpu.VMEM((2, _BK), jnp.int32),
            pltpu.VMEM((2, _BK), jnp.int32),
            pltpu.VMEM((_BK, HALF), jnp.float32),
            pltpu.VMEM((_BK, HALF), jnp.float32),
            pltpu.VMEM((25, HALF), jnp.float32),
            pltpu.VMEM_SHARED((NN, HALF), jnp.float32),
            pltpu.SemaphoreType.DMA,
            pltpu.SemaphoreType.DMA,
        ],
    )


def _agg_body(hs_lo_hbm, hs_hi_hbm, eb_hbm, out_hbm,
              idx0, idx1, rows0, rows1, zbuf, acc, sem0, sem1):
    c = lax.axis_index("c")
    s = lax.axis_index("s")

    _zero_fill(zbuf, 25, HALF)

    def zcopy(i, _):
        pltpu.sync_copy(zbuf, acc.at[pl.ds(s * ROWS_PER_TILE + i * 25, 25)])
        return 0

    lax.fori_loop(0, 25, zcopy, 0)
    plsc.subcore_barrier()

    base = s * _AGG_BPT

    def run_edges(hs_ref):
        bufs = ((idx0, rows0, sem0), (idx1, rows1, sem1))

        def start(i, b):
            idx, rows, sem = b
            pltpu.sync_copy(eb_hbm.at[base + i], idx)
            pltpu.async_copy(hs_ref.at[idx.at[0]], rows, sem)

        def finish(b):
            idx, rows, sem = b
            pltpu.make_async_copy(hs_ref.at[idx.at[0]], rows, sem).wait()
            pltpu.sync_copy(rows, acc.at[idx.at[1]], add=True)

        start(0, bufs[0])
        start(1, bufs[1])

        def body(p, _):
            i = p * 2
            finish(bufs[0])

            @pl.when(i + 2 < _AGG_BPT)
            def _():
                start(i + 2, bufs[0])

            finish(bufs[1])

            @pl.when(i + 3 < _AGG_BPT)
            def _():
                start(i + 3, bufs[1])

            return 0

        lax.fori_loop(0, _AGG_BPT // 2, body, 0)

    pl.when(c == 0)(lambda: run_edges(hs_lo_hbm))
    pl.when(c == 1)(lambda: run_edges(hs_hi_hbm))
    plsc.subcore_barrier()

    pltpu.sync_copy(acc.at[pl.ds(s * ROWS_PER_TILE, ROWS_PER_TILE)],
                    out_hbm.at[c].at[s])


_BN = 2000


def _a0_body(deg_ref, x_ref, w_ref, lo_ref, hi_ref, dinv_ref):
    d = deg_ref[0, :, 0] + deg_ref[1, :, 0] + 1.0
    dv = lax.rsqrt(d)
    hw = jnp.dot(x_ref[...], w_ref[...], preferred_element_type=jnp.float32)
    hs = hw * dv[:, None]
    lo_ref[...] = hs[:, :HALF]
    hi_ref[...] = hs[:, HALF:]
    dinv_ref[...] = dv[:, None]


def _layer0(degs, x, w0):
    return pl.pallas_call(
        _a0_body,
        grid=(NN // _BN,),
        in_specs=[
            pl.BlockSpec((NC, _BN, 16), lambda i: (0, i, 0)),
            pl.BlockSpec((_BN, F_IN), lambda i: (i, 0)),
            pl.BlockSpec((F_IN, HID), lambda i: (0, 0)),
        ],
        out_specs=[
            pl.BlockSpec((_BN, HALF), lambda i: (i, 0)),
            pl.BlockSpec((_BN, HALF), lambda i: (i, 0)),
            pl.BlockSpec((_BN, 1), lambda i: (i, 0)),
        ],
        out_shape=[
            jax.ShapeDtypeStruct((NN, HALF), jnp.float32),
            jax.ShapeDtypeStruct((NN, HALF), jnp.float32),
            jax.ShapeDtypeStruct((NN, 1), jnp.float32),
        ],
    )(degs, x, w0)


def _amid_body(agg_ref, lo_ref, hi_ref, dinv_ref, b_ref, w_ref,
               olo_ref, ohi_ref):
    dv = dinv_ref[...]
    lo = jax.nn.relu((agg_ref[0] + lo_ref[...]) * dv + b_ref[0, :HALF][None, :])
    hi = jax.nn.relu((agg_ref[1] + hi_ref[...]) * dv + b_ref[0, HALF:][None, :])
    h = jnp.concatenate([lo, hi], axis=1)
    hs = jnp.dot(h, w_ref[...], preferred_element_type=jnp.float32) * dv
    olo_ref[...] = hs[:, :HALF]
    ohi_ref[...] = hs[:, HALF:]


def _layer_mid(agg, hs_lo, hs_hi, dinv, b_prev, w):
    return pl.pallas_call(
        _amid_body,
        grid=(NN // _BN,),
        in_specs=[
            pl.BlockSpec((NC, _BN, HALF), lambda i: (0, i, 0)),
            pl.BlockSpec((_BN, HALF), lambda i: (i, 0)),
            pl.BlockSpec((_BN, HALF), lambda i: (i, 0)),
            pl.BlockSpec((_BN, 1), lambda i: (i, 0)),
            pl.BlockSpec((1, HID), lambda i: (0, 0)),
            pl.BlockSpec((HID, HID), lambda i: (0, 0)),
        ],
        out_specs=[
            pl.BlockSpec((_BN, HALF), lambda i: (i, 0)),
            pl.BlockSpec((_BN, HALF), lambda i: (i, 0)),
        ],
        out_shape=[
            jax.ShapeDtypeStruct((NN, HALF), jnp.float32),
            jax.ShapeDtypeStruct((NN, HALF), jnp.float32),
        ],
    )(agg, hs_lo, hs_hi, dinv, b_prev, w)


def _a3_body(agg_ref, lo_ref, hi_ref, dinv_ref, b_ref, wl_ref, bl_ref,
             out_ref):
    dv = dinv_ref[...]
    lo = jax.nn.relu((agg_ref[0] + lo_ref[...]) * dv + b_ref[0, :HALF][None, :])
    hi = jax.nn.relu((agg_ref[1] + hi_ref[...]) * dv + b_ref[0, HALF:][None, :])
    h = jnp.concatenate([lo, hi], axis=1)
    out_ref[...] = (
        jnp.dot(h, wl_ref[...], preferred_element_type=jnp.float32)
        + bl_ref[0, 0]
    )


def _layer_final(agg, hs_lo, hs_hi, dinv, b2, wl, bl):
    return pl.pallas_call(
        _a3_body,
        grid=(NN // _BN,),
        in_specs=[
            pl.BlockSpec((NC, _BN, HALF), lambda i: (0, i, 0)),
            pl.BlockSpec((_BN, HALF), lambda i: (i, 0)),
            pl.BlockSpec((_BN, HALF), lambda i: (i, 0)),
            pl.BlockSpec((_BN, 1), lambda i: (i, 0)),
            pl.BlockSpec((1, HID), lambda i: (0, 0)),
            pl.BlockSpec((HID, 1), lambda i: (0, 0)),
            pl.BlockSpec((1, 1), lambda i: (0, 0)),
        ],
        out_specs=pl.BlockSpec((_BN, 1), lambda i: (i, 0)),
        out_shape=jax.ShapeDtypeStruct((NN, 1), jnp.float32),
    )(agg, hs_lo, hs_hi, dinv, b2, wl, bl)


def kernel(x, edge_index, W0, b0, W1, b1, W2, b2, Wl, bl):
    src = edge_index[0].astype(jnp.int32)
    dst = edge_index[1].astype(jnp.int32)
    eb = jnp.stack([src.reshape(_NBLK, _BK), dst.reshape(_NBLK, _BK)], axis=1)

    degs = _get_deg_kernel()(eb).reshape(NC, NN, 16)
    hs_lo, hs_hi, dinv = _layer0(degs, x, W0)

    agg = _get_agg_kernel()(hs_lo, hs_hi, eb).reshape(NC, NN, HALF)
    hs_lo, hs_hi = _layer_mid(agg, hs_lo, hs_hi, dinv, b0.reshape(1, -1), W1)

    agg = _get_agg_kernel()(hs_lo, hs_hi, eb).reshape(NC, NN, HALF)
    hs_lo, hs_hi = _layer_mid(agg, hs_lo, hs_hi, dinv, b1.reshape(1, -1), W2)

    agg = _get_agg_kernel()(hs_lo, hs_hi, eb).reshape(NC, NN, HALF)
    out = _layer_final(agg, hs_lo, hs_hi, dinv, b2.reshape(1, -1), Wl,
                       bl.reshape(1, 1))
    return out.reshape(-1)
